# Initial kernel scaffold; baseline (speedup 1.0000x reference)
import jax
import jax.numpy as jnp
from jax.experimental import pallas as pl


def kernel(raw_images, image_sizes, featurized_images, rpn_conv_w, rpn_conv_b, rpn_obj_w, rpn_obj_b, rpn_delta_w, rpn_delta_b, w_cls, b_cls, w_bbox, b_bbox):
    raise NotImplementedError("write your pallas kernel here")


# speedup vs baseline: 13.3931x; 13.3931x over previous
import functools
import math

import jax
import jax.numpy as jnp
import numpy as np
from jax import lax
from jax.experimental import pallas as pl
from jax.experimental.pallas import tpu as pltpu

_SIZES = [32.0, 64.0, 128.0, 256.0, 512.0]
_RATIOS = [0.5, 1.0, 2.0]
_STRIDE = 16
_PRE_NMS = 6000
_POST_NMS = 1000
_RPN_NMS_T = 0.7
_NUM_CLASSES = 1600
_FINAL_K = 100
_FINAL_NMS_T = 0.5
_POOL_S = 4
_SCALE_CLAMP = math.log(1000.0 / 16.0)
_NEG = -1e30
_HP = jax.lax.Precision.HIGHEST


def _np_anchors(Hf, Wf, stride):
    sizes = np.asarray(_SIZES, np.float32)
    ratios = np.asarray(_RATIOS, np.float32)
    ws = (sizes[:, None] / np.sqrt(ratios)[None, :]).reshape(-1)
    hs = (sizes[:, None] * np.sqrt(ratios)[None, :]).reshape(-1)
    sx = (np.arange(Wf, dtype=np.float32) + np.float32(0.5)) * np.float32(stride)
    sy = (np.arange(Hf, dtype=np.float32) + np.float32(0.5)) * np.float32(stride)
    cy, cx = np.meshgrid(sy, sx, indexing="ij")
    cx = cx[:, :, None].astype(np.float32)
    cy = cy[:, :, None].astype(np.float32)
    x1 = cx - np.float32(0.5) * ws
    y1 = cy - np.float32(0.5) * hs
    x2 = cx + np.float32(0.5) * ws
    y2 = cy + np.float32(0.5) * hs
    a = np.stack([x1, y1, x2, y2], axis=-1).reshape(-1, 4).astype(np.float32)
    w = a[:, 2] - a[:, 0]
    h = a[:, 3] - a[:, 1]
    acx = a[:, 0] + np.float32(0.5) * w
    acy = a[:, 1] + np.float32(0.5) * h
    return w, h, acx, acy


def _k1_body(x_ref, w_ref, wod_ref, brpn_ref, bod_ref, od_ref, acc_ref):
    k = pl.program_id(0)
    nk = pl.num_programs(0)

    @pl.when(k == 0)
    def _():
        acc_ref[...] = jnp.zeros_like(acc_ref)

    acc_ref[...] += jax.lax.dot_general(
        x_ref[...], w_ref[...], (((1,), (0,)), ((), ())),
        preferred_element_type=jnp.float32)

    @pl.when(k == nk - 1)
    def _():
        t = jnp.maximum(acc_ref[...] + brpn_ref[...], 0.0)
        od_ref[...] = jax.lax.dot_general(
            t, wod_ref[...], (((1,), (0,)), ((), ())),
            preferred_element_type=jnp.float32) + bod_ref[...]


def _k1(x_col, w_mat, wod, brpn, bod):
    nk = x_col.shape[1] // 1024
    return pl.pallas_call(
        _k1_body,
        grid=(nk,),
        in_specs=[
            pl.BlockSpec((1024, 1024), lambda k: (0, k)),
            pl.BlockSpec((1024, 1024), lambda k: (k, 0)),
            pl.BlockSpec((1024, 128), lambda k: (0, 0)),
            pl.BlockSpec((1, 1024), lambda k: (0, 0)),
            pl.BlockSpec((1, 128), lambda k: (0, 0)),
        ],
        out_specs=pl.BlockSpec((1024, 128), lambda k: (0, 0)),
        out_shape=jax.ShapeDtypeStruct((1024, 128), jnp.float32),
        scratch_shapes=[pltpu.VMEM((1024, 1024), jnp.float32)],
    )(x_col, w_mat, wod, brpn, bod)


def _k3_body(obj_ref, d0_ref, d1_ref, d2_ref, d3_ref,
             aw_ref, ah_ref, acx_ref, acy_ref, hw_ref,
             px1_ref, py1_ref, px2_ref, py2_ref):
    h_img = hw_ref[0, 0]
    w_img = hw_ref[0, 1]
    obj = obj_ref[...]
    aw = aw_ref[...]
    ah = ah_ref[...]
    acx = acx_ref[...]
    acy = acy_ref[...]
    dw = jnp.minimum(d2_ref[...], _SCALE_CLAMP)
    dh = jnp.minimum(d3_ref[...], _SCALE_CLAMP)
    pcx = d0_ref[...] * aw + acx
    pcy = d1_ref[...] * ah + acy
    pw = jnp.exp(dw) * aw
    ph = jnp.exp(dh) * ah
    x1 = jnp.minimum(jnp.maximum(pcx - 0.5 * pw, 0.0), w_img)
    y1 = jnp.minimum(jnp.maximum(pcy - 0.5 * ph, 0.0), h_img)
    x2 = jnp.minimum(jnp.maximum(pcx + 0.5 * pw, 0.0), w_img)
    y2 = jnp.minimum(jnp.maximum(pcy + 0.5 * ph, 0.0), h_img)
    areas = (x2 - x1) * (y2 - y1)

    sbits = jax.lax.bitcast_convert_type(obj, jnp.int32)
    int_min = jnp.int32(-2147483648)
    key = jnp.where(sbits >= 0, sbits, int_min - sbits)

    def bisect(_, lohi):
        lo, hi = lohi
        mid = (lo >> 1) + (hi >> 1) + (lo & hi & 1)
        cnt = jnp.sum((key >= mid).astype(jnp.int32))
        ge = cnt >= _PRE_NMS
        return jnp.where(ge, mid, lo), jnp.where(ge, hi, mid)

    lo, _ = lax.fori_loop(0, 36, bisect, (int_min, jnp.int32(2147483647)))
    eligible = key >= lo

    iota = (lax.broadcasted_iota(jnp.int32, obj.shape, 0) * 128
            + lax.broadcasted_iota(jnp.int32, obj.shape, 1))
    kiota = (lax.broadcasted_iota(jnp.int32, (8, 128), 0) * 128
             + lax.broadcasted_iota(jnp.int32, (8, 128), 1))
    zk = jnp.zeros((8, 128), jnp.float32)
    s0 = jnp.where(eligible, obj, _NEG)

    def nms_step(k, carry):
        s, i0, kx1, ky1, kx2, ky2 = carry
        m = jnp.max(s)
        isel = jnp.min(jnp.where(s == m, iota, jnp.int32(2147483647)))
        isel = jnp.where(m > -1e29, isel, i0)
        i0 = jnp.where(k == 0, isel, i0)
        onehot = iota == isel
        bx1 = jnp.sum(jnp.where(onehot, x1, 0.0))
        by1 = jnp.sum(jnp.where(onehot, y1, 0.0))
        bx2 = jnp.sum(jnp.where(onehot, x2, 0.0))
        by2 = jnp.sum(jnp.where(onehot, y2, 0.0))
        barea = jnp.sum(jnp.where(onehot, areas, 0.0))
        xx1 = jnp.maximum(bx1, x1)
        yy1 = jnp.maximum(by1, y1)
        xx2 = jnp.minimum(bx2, x2)
        yy2 = jnp.minimum(by2, y2)
        inter = jnp.maximum(xx2 - xx1, 0.0) * jnp.maximum(yy2 - yy1, 0.0)
        iou = inter / (barea + areas - inter + 1e-9)
        s = jnp.where(iou >= _RPN_NMS_T, _NEG, s)
        s = jnp.where(onehot, _NEG, s)
        ksel = kiota == k
        kx1 = jnp.where(ksel, bx1, kx1)
        ky1 = jnp.where(ksel, by1, ky1)
        kx2 = jnp.where(ksel, bx2, kx2)
        ky2 = jnp.where(ksel, by2, ky2)
        return s, i0, kx1, ky1, kx2, ky2

    _, _, kx1, ky1, kx2, ky2 = lax.fori_loop(
        0, _POST_NMS, nms_step, (s0, jnp.int32(0), zk, zk, zk, zk))
    px1_ref[...] = kx1
    py1_ref[...] = ky1
    px2_ref[...] = kx2
    py2_ref[...] = ky2


def _k3(obj, d0, d1, d2, d3, aw, ah, acx, acy, hw):
    o = jax.ShapeDtypeStruct((8, 128), jnp.float32)
    return pl.pallas_call(
        _k3_body,
        in_specs=[pl.BlockSpec(memory_space=pltpu.VMEM)] * 9
        + [pl.BlockSpec(memory_space=pltpu.SMEM)],
        out_shape=(o, o, o, o),
    )(obj, d0, d1, d2, d3, aw, ah, acx, acy, hw)


def _k4_body(x1_ref, y1_ref, x2_ref, y2_ref, feat_ref, out_ref):
    x1 = x1_ref[...]
    y1 = y1_ref[...]
    x2 = x2_ref[...]
    y2 = y2_ref[...]
    riota = lax.broadcasted_iota(jnp.int32, (1, 1024), 1)
    ycol = (riota // 32).astype(jnp.float32)
    xcol = (riota % 32).astype(jnp.float32)

    def factors(loc, hic, col):
        out = []
        for j in range(_POOL_S):
            g = loc + np.float32((j + 0.5) / _POOL_S) * (hic - loc)
            c = g * np.float32(1.0 / _STRIDE) - 0.5
            c = jnp.minimum(jnp.maximum(c, 0.0), 31.0)
            c0 = jnp.floor(c)
            wc = c - c0
            c1 = jnp.minimum(c0 + 1.0, 31.0)
            out.append((col == c0) * (1.0 - wc) + (col == c1) * wc)
        return out

    rx = factors(x1, x2, xcol)
    ry = factors(y1, y2, ycol)
    wmat = jnp.zeros((128, 1024), jnp.float32)
    for iy in range(_POOL_S):
        for ix in range(_POOL_S):
            wmat = wmat + ry[iy] * rx[ix]
    out_ref[...] = jax.lax.dot_general(
        wmat, feat_ref[...], (((1,), (0,)), ((), ())),
        preferred_element_type=jnp.float32, precision=_HP) * np.float32(1.0 / 16.0)


def _k4(x1c, y1c, x2c, y2c, featpos):
    return pl.pallas_call(
        _k4_body,
        grid=(8,),
        in_specs=[
            pl.BlockSpec((128, 1), lambda i: (i, 0)),
            pl.BlockSpec((128, 1), lambda i: (i, 0)),
            pl.BlockSpec((128, 1), lambda i: (i, 0)),
            pl.BlockSpec((128, 1), lambda i: (i, 0)),
            pl.BlockSpec((1024, 1024), lambda i: (0, 0)),
        ],
        out_specs=pl.BlockSpec((128, 1024), lambda i: (i, 0)),
        out_shape=jax.ShapeDtypeStruct((1024, 1024), jnp.float32),
    )(x1c, y1c, x2c, y2c, featpos)


def _k5a_body(p_ref, w_ref, b_ref, probs_ref, sc2_ref, d2_ref):
    logits = jax.lax.dot_general(
        p_ref[...], w_ref[...], (((1,), (0,)), ((), ())),
        preferred_element_type=jnp.float32) + b_ref[...]
    ci = lax.broadcasted_iota(jnp.int32, logits.shape, 1)
    valid = ci < (_NUM_CLASSES + 1)
    lm = jnp.where(valid, logits, _NEG)
    mx = jnp.max(lm, axis=1, keepdims=True)
    e = jnp.exp(lm - mx)
    probs = e / jnp.sum(e, axis=1, keepdims=True)
    probs_ref[...] = probs
    sc2_ref[...] = jnp.max(
        jnp.where(ci < _NUM_CLASSES, probs, -1.0), axis=1, keepdims=True)
    d2_ref[...] = logits[:, 1664:1792]


def _k5a(pooled, w_all, b_all):
    return pl.pallas_call(
        _k5a_body,
        grid=(8,),
        in_specs=[
            pl.BlockSpec((128, 1024), lambda i: (i, 0)),
            pl.BlockSpec((1024, 1792), lambda i: (0, 0)),
            pl.BlockSpec((1, 1792), lambda i: (0, 0)),
        ],
        out_specs=(
            pl.BlockSpec((128, 1792), lambda i: (i, 0)),
            pl.BlockSpec((128, 1), lambda i: (i, 0)),
            pl.BlockSpec((128, 128), lambda i: (i, 0)),
        ),
        out_shape=(
            jax.ShapeDtypeStruct((1024, 1792), jnp.float32),
            jax.ShapeDtypeStruct((1024, 1), jnp.float32),
            jax.ShapeDtypeStruct((1024, 128), jnp.float32),
        ),
    )(pooled, w_all, b_all)


def _k5b_body(sc_ref, dx_ref, dy_ref, dw_ref, dh_ref,
              px1_ref, py1_ref, px2_ref, py2_ref, hw_ref,
              sel_ref, kx1_ref, ky1_ref, kx2_ref, ky2_ref):
    h_img = hw_ref[0, 0]
    w_img = hw_ref[0, 1]
    w = px2_ref[...] - px1_ref[...]
    h = py2_ref[...] - py1_ref[...]
    cx = px1_ref[...] + 0.5 * w
    cy = py1_ref[...] + 0.5 * h
    dw = jnp.minimum(dw_ref[...], _SCALE_CLAMP)
    dh = jnp.minimum(dh_ref[...], _SCALE_CLAMP)
    pcx = dx_ref[...] * w + cx
    pcy = dy_ref[...] * h + cy
    pw = jnp.exp(dw) * w
    ph = jnp.exp(dh) * h
    x1 = jnp.minimum(jnp.maximum(pcx - 0.5 * pw, 0.0), w_img)
    y1 = jnp.minimum(jnp.maximum(pcy - 0.5 * ph, 0.0), h_img)
    x2 = jnp.minimum(jnp.maximum(pcx + 0.5 * pw, 0.0), w_img)
    y2 = jnp.minimum(jnp.maximum(pcy + 0.5 * ph, 0.0), h_img)
    areas = (x2 - x1) * (y2 - y1)

    niota = (lax.broadcasted_iota(jnp.int32, (8, 128), 0) * 128
             + lax.broadcasted_iota(jnp.int32, (8, 128), 1))
    s0 = jnp.where(niota < _POST_NMS, sc_ref[...], _NEG)
    zk = jnp.zeros((8, 128), jnp.float32)

    def nms_step(k, carry):
        s, sel, kx1, ky1, kx2, ky2 = carry
        m = jnp.max(s)
        isel = jnp.min(jnp.where(s == m, niota, jnp.int32(2147483647)))
        onehot = niota == isel
        bx1 = jnp.sum(jnp.where(onehot, x1, 0.0))
        by1 = jnp.sum(jnp.where(onehot, y1, 0.0))
        bx2 = jnp.sum(jnp.where(onehot, x2, 0.0))
        by2 = jnp.sum(jnp.where(onehot, y2, 0.0))
        barea = jnp.sum(jnp.where(onehot, areas, 0.0))
        xx1 = jnp.maximum(bx1, x1)
        yy1 = jnp.maximum(by1, y1)
        xx2 = jnp.minimum(bx2, x2)
        yy2 = jnp.minimum(by2, y2)
        inter = jnp.maximum(xx2 - xx1, 0.0) * jnp.maximum(yy2 - yy1, 0.0)
        iou = inter / (barea + areas - inter + 1e-9)
        s = jnp.where(iou >= _FINAL_NMS_T, _NEG, s)
        s = jnp.where(onehot, _NEG, s)
        ksel = niota == k
        sel = jnp.where(ksel, isel, sel)
        kx1 = jnp.where(ksel, bx1, kx1)
        ky1 = jnp.where(ksel, by1, ky1)
        kx2 = jnp.where(ksel, bx2, kx2)
        ky2 = jnp.where(ksel, by2, ky2)
        return s, sel, kx1, ky1, kx2, ky2

    _, sel, kx1, ky1, kx2, ky2 = lax.fori_loop(
        0, _FINAL_K, nms_step,
        (s0, jnp.zeros((8, 128), jnp.int32), zk, zk, zk, zk))
    sel_ref[...] = sel
    kx1_ref[...] = kx1
    ky1_ref[...] = ky1
    kx2_ref[...] = kx2
    ky2_ref[...] = ky2


def _k5b(sc2v, dxv, dyv, dwv, dhv, px1, py1, px2, py2, hw):
    of = jax.ShapeDtypeStruct((8, 128), jnp.float32)
    oi = jax.ShapeDtypeStruct((8, 128), jnp.int32)
    return pl.pallas_call(
        _k5b_body,
        in_specs=[pl.BlockSpec(memory_space=pltpu.VMEM)] * 9
        + [pl.BlockSpec(memory_space=pltpu.SMEM)],
        out_shape=(oi, of, of, of, of),
    )(sc2v, dxv, dyv, dwv, dhv, px1, py1, px2, py2, hw)


def _k6_body(sel_ref, pooled_ref, probs_ref, o1_ref, o2_ref):
    niota = lax.broadcasted_iota(jnp.int32, (128, 1024), 1)
    g = (niota == sel_ref[...]).astype(jnp.float32)
    o1_ref[...] = jax.lax.dot_general(
        g, pooled_ref[...], (((1,), (0,)), ((), ())),
        preferred_element_type=jnp.float32, precision=_HP)
    o2_ref[...] = jax.lax.dot_general(
        g, probs_ref[...], (((1,), (0,)), ((), ())),
        preferred_element_type=jnp.float32, precision=_HP)


def _k6(selcol, pooled, probs):
    return pl.pallas_call(
        _k6_body,
        out_shape=(
            jax.ShapeDtypeStruct((128, 1024), jnp.float32),
            jax.ShapeDtypeStruct((128, 1792), jnp.float32),
        ),
    )(selcol, pooled, probs)


def kernel(raw_images, image_sizes, featurized_images, rpn_conv_w, rpn_conv_b,
           rpn_obj_w, rpn_obj_b, rpn_delta_w, rpn_delta_b, w_cls, b_cls,
           w_bbox, b_bbox):
    del raw_images
    Hf, Wf = featurized_images.shape[2], featurized_images.shape[3]
    A = len(_SIZES) * len(_RATIOS)
    npos = Hf * Wf
    nanch = npos * A

    f = featurized_images[0]
    fh = jnp.transpose(f, (1, 2, 0))
    fp = jnp.pad(fh, ((1, 1), (1, 1), (0, 0)))
    cols = [fp[dy:dy + Hf, dx:dx + Wf, :] for dy in range(3) for dx in range(3)]
    x_col = jnp.stack(cols, axis=2).reshape(npos, 9 * 1024)
    w_mat = jnp.transpose(rpn_conv_w, (2, 3, 1, 0)).reshape(9 * 1024, 1024)
    wod = jnp.concatenate([
        rpn_obj_w.reshape(A, 1024).T,
        rpn_delta_w.reshape(4 * A, 1024).T,
        jnp.zeros((1024, 128 - 5 * A), jnp.float32)], axis=1)
    bod = jnp.concatenate([
        rpn_obj_b, rpn_delta_b, jnp.zeros((128 - 5 * A,), jnp.float32)
    ]).reshape(1, 128)
    od = _k1(x_col, w_mat, wod, rpn_conv_b.reshape(1, 1024), bod)

    obj = od[:, :A].reshape(nanch // 128, 128)
    dmat = od[:, A:5 * A].reshape(npos, A, 4)
    dsplit = [dmat[:, :, j].reshape(nanch // 128, 128) for j in range(4)]

    aw, ah, acx, acy = _np_anchors(Hf, Wf, _STRIDE)
    awj, ahj, acxj, acyj = (jnp.asarray(v.reshape(nanch // 128, 128))
                            for v in (aw, ah, acx, acy))
    hw = image_sizes.astype(jnp.float32).reshape(1, 2)

    px1, py1, px2, py2 = _k3(obj, *dsplit, awj, ahj, acxj, acyj, hw)

    featpos = f.reshape(1024, npos).T
    pcols = [v.reshape(1024, 1) for v in (px1, py1, px2, py2)]
    pooled = _k4(*pcols, featpos)

    w_all = jnp.concatenate([
        w_cls, jnp.zeros((1024, 1664 - (_NUM_CLASSES + 1)), jnp.float32),
        w_bbox, jnp.zeros((1024, 124), jnp.float32)], axis=1)
    b_all = jnp.concatenate([
        b_cls, jnp.zeros((1664 - (_NUM_CLASSES + 1),), jnp.float32),
        b_bbox, jnp.zeros((124,), jnp.float32)]).reshape(1, 1792)
    probs, sc2col, d2col = _k5a(pooled, w_all, b_all)

    sc2v = sc2col.reshape(8, 128)
    dvs = [d2col[:, j].reshape(8, 128) for j in range(4)]
    sel, kx1, ky1, kx2, ky2 = _k5b(sc2v, *dvs, px1, py1, px2, py2, hw)

    selcol = sel.reshape(1024)[:128].reshape(128, 1)
    o1, o2 = _k6(selcol, pooled, probs)

    out_pooled = o1[:_FINAL_K, :]
    out_boxes = jnp.stack([v.reshape(1024)[:_FINAL_K]
                           for v in (kx1, ky1, kx2, ky2)], axis=1)
    out_probs = o2[:_FINAL_K, :_NUM_CLASSES]
    return out_pooled, out_boxes, out_probs
